# baseline (speedup 1.0000x reference)
import jax
import jax.numpy as jnp
from jax.experimental import pallas as pl
from jax.experimental.pallas import tpu as pltpu

HEIGHT = 32
WIDTH = 32
D_MODEL = 768


B_BLK = 4


def _add_pos_body(x_ref, row_ref, col_ref, out_ref):
    pos = row_ref[...][None, :, None, :] + col_ref[...][None, None, :, :]
    out_ref[...] = x_ref[...] + pos


def kernel(x, row_embed, col_embed):
    batch, seq_len, d = x.shape
    x4 = x.reshape(batch, HEIGHT, WIDTH, d)
    out = pl.pallas_call(
        _add_pos_body,
        grid=(batch // B_BLK,),
        in_specs=[
            pl.BlockSpec((B_BLK, HEIGHT, WIDTH, d), lambda b: (b, 0, 0, 0)),
            pl.BlockSpec((HEIGHT, d), lambda b: (0, 0)),
            pl.BlockSpec((WIDTH, d), lambda b: (0, 0)),
        ],
        out_specs=pl.BlockSpec((B_BLK, HEIGHT, WIDTH, d), lambda b: (b, 0, 0, 0)),
        out_shape=jax.ShapeDtypeStruct((batch, HEIGHT, WIDTH, d), x.dtype),
        compiler_params=pltpu.CompilerParams(vmem_limit_bytes=120 * 1024 * 1024),
    )(x4, row_embed, col_embed)
    return out.reshape(batch, seq_len, d)
